# speedup vs baseline: 1.0851x; 1.0851x over previous
import jax
import jax.numpy as jnp
from jax.experimental import pallas as pl
from jax.experimental.pallas import tpu as pltpu


def _hconv(x, ni, ei, W, b, n_nodes, n_hedges):
    xw = x @ W + b
    ones = jnp.ones(ni.shape[0], dtype=x.dtype)
    D = jax.ops.segment_sum(ones, ni, num_segments=n_nodes)
    B = jax.ops.segment_sum(ones, ei, num_segments=n_hedges)
    Dinv = jnp.where(D > 0, 1.0 / D, 0.0)
    Binv = jnp.where(B > 0, 1.0 / B, 0.0)
    e_msg = jax.ops.segment_sum(xw[ni], ei, num_segments=n_hedges) * Binv[:, None]
    out = jax.ops.segment_sum(e_msg[ei], ni, num_segments=n_nodes) * Dinv[:, None]
    return out


def _head_kernel(h_ref, w1_ref, b1_ref, w2_ref, b2_ref, out_ref):
    h = h_ref[...]
    z = jnp.maximum(h @ w1_ref[...] + b1_ref[...][None, :], 0.0)
    logits = z @ w2_ref[...] + b2_ref[...][None, :]
    m = jnp.max(logits, axis=1, keepdims=True)
    s = jnp.log(jnp.sum(jnp.exp(logits - m), axis=1, keepdims=True)) + m
    out_ref[...] = logits - s


def _head(h, W1, b1, W2, b2):
    return pl.pallas_call(
        _head_kernel,
        out_shape=jax.ShapeDtypeStruct((h.shape[0], 2), jnp.float32),
    )(h, W1, b1, W2, b2)


def kernel(x, edge_index, marks, edge_x, edge_marks,
           Wn0, bn0, We0, be0, Wn1, bn1, We1, be1,
           Wn2, bn2, We2, be2, Wn3, bn3, We3, be3,
           W1, b1, W2, b2):
    N = x.shape[0]
    He = edge_x.shape[0]
    ni = edge_index[0]
    ei = edge_index[1]
    Wns = [(Wn0, bn0), (Wn1, bn1), (Wn2, bn2), (Wn3, bn3)]
    Wes = [(We0, be0), (We1, be1), (We2, be2), (We3, be3)]
    all_x, all_e = [], []
    cur, cure = x, edge_x
    for (Wn, bn), (We, be) in zip(Wns, Wes):
        cur = jax.nn.relu(_hconv(cur, ni, ei, Wn, bn, N, He))
        all_x.append(cur)
        cure = jax.nn.relu(_hconv(cure, ei, ni, We, be, He, N))
        all_e.append(cure)
    xc = jnp.concatenate(all_x, axis=1)[marks]
    ec = jnp.concatenate(all_e, axis=1)
    e1 = ec[edge_marks]
    e2 = ec[edge_marks + 1]
    ex = jnp.concatenate([jnp.minimum(e1, e2), jnp.maximum(e1, e2)], axis=1)
    h = jnp.concatenate([ex, xc], axis=1)
    return _head(h, W1, b1, W2, b2)


# speedup vs baseline: 5.1396x; 4.7366x over previous
import functools

import jax
import jax.numpy as jnp
from jax import lax
from jax.experimental import pallas as pl
from jax.experimental.pallas import tpu as pltpu
from jax.experimental.pallas import tpu_sc as plsc

NC = 2
NS = 16
R = 50000
RR = 50048
E = 800000
EP = 819200
NROWS = EP // 128
TROWS = NROWS // NS
CHROWS = 4
NCHUNK = TROWS // CHROWS
RPT = RR // NS

_mesh = plsc.VectorSubcoreMesh(
    core_axis_name="c", subcore_axis_name="s", num_cores=NC, num_subcores=NS)


def _spmm_body(table, srcs, dsts, zrows, out, src_v, dst_v, rows_v, acc, sem):
    c = lax.axis_index("c")
    s = lax.axis_index("s")
    pltpu.sync_copy(zrows, acc.at[pl.ds(s * RPT, RPT)])
    plsc.subcore_barrier()

    t0 = s * TROWS

    def chunk(k, _):
        r0 = t0 + k * CHROWS
        pltpu.sync_copy(srcs.at[c, pl.ds(r0, CHROWS)], src_v)
        pltpu.sync_copy(dsts.at[pl.ds(r0, CHROWS)], dst_v)
        cps = [
            pltpu.async_copy(table.at[src_v.at[j]],
                             rows_v.at[pl.ds(j * 128, 128)], sem)
            for j in range(CHROWS)
        ]
        for cp in cps:
            cp.wait()
        for j in range(CHROWS):
            pltpu.sync_copy(rows_v.at[pl.ds(j * 128, 128)],
                            acc.at[dst_v.at[j]], add=True)
        return 0

    lax.fori_loop(0, NCHUNK, chunk, 0)
    plsc.subcore_barrier()
    pltpu.sync_copy(acc.at[pl.ds(s * RPT, RPT)],
                    out.at[c, pl.ds(s * RPT, RPT)])


def _make_spmm(fh):
    return pl.kernel(
        functools.partial(_spmm_body),
        out_type=jax.ShapeDtypeStruct((NC, RR, fh), jnp.float32),
        mesh=_mesh,
        compiler_params=pltpu.CompilerParams(use_tc_tiling_on_sc=False),
        scratch_types=[
            pltpu.VMEM((CHROWS, 128), jnp.int32),
            pltpu.VMEM((CHROWS, 128), jnp.int32),
            pltpu.VMEM((CHROWS * 128, fh), jnp.float32),
            pltpu.VMEM_SHARED((RR, fh), jnp.float32),
            pltpu.SemaphoreType.DMA,
        ],
    )


_spmm32 = _make_spmm(32)
_spmm16 = _make_spmm(16)


def _deg_body(dsts, ones_h, zrows, out, dst_v, ones_v, acc, sem):
    c = lax.axis_index("c")
    s = lax.axis_index("s")
    pltpu.sync_copy(zrows, acc.at[pl.ds(s * RPT, RPT)])
    pltpu.sync_copy(ones_h, ones_v)
    plsc.subcore_barrier()

    t0 = s * TROWS

    def chunk(k, _):
        r0 = t0 + k * CHROWS
        pltpu.sync_copy(dsts.at[c, pl.ds(r0, CHROWS)], dst_v)
        for j in range(CHROWS):
            pltpu.sync_copy(ones_v, acc.at[dst_v.at[j]], add=True)
        return 0

    lax.fori_loop(0, NCHUNK, chunk, 0)
    plsc.subcore_barrier()
    pltpu.sync_copy(acc.at[pl.ds(s * RPT, RPT)],
                    out.at[c, pl.ds(s * RPT, RPT)])


_deg_kernel = pl.kernel(
    _deg_body,
    out_type=jax.ShapeDtypeStruct((NC, RR, 16), jnp.float32),
    mesh=_mesh,
    compiler_params=pltpu.CompilerParams(use_tc_tiling_on_sc=False),
    scratch_types=[
        pltpu.VMEM((CHROWS, 128), jnp.int32),
        pltpu.VMEM((128, 16), jnp.float32),
        pltpu.VMEM_SHARED((RR, 16), jnp.float32),
        pltpu.SemaphoreType.DMA,
    ],
)


def _head_kernel_body(h_ref, w1_ref, b1_ref, w2_ref, b2_ref, out_ref):
    h = h_ref[...]
    z = jnp.maximum(h @ w1_ref[...] + b1_ref[...][None, :], 0.0)
    logits = z @ w2_ref[...] + b2_ref[...][None, :]
    m = jnp.max(logits, axis=1, keepdims=True)
    sm = jnp.log(jnp.sum(jnp.exp(logits - m), axis=1, keepdims=True)) + m
    out_ref[...] = logits - sm


def _head(h, W1, b1, W2, b2):
    return pl.pallas_call(
        _head_kernel_body,
        out_shape=jax.ShapeDtypeStruct((h.shape[0], 2), jnp.float32),
    )(h, W1, b1, W2, b2)


def _split(t):
    n, f = t.shape
    th = t.reshape(n, 2, f // 2).transpose(1, 0, 2)
    th = jnp.pad(th, ((0, 0), (0, RR - n), (0, 0)))
    return th.reshape(2 * RR, f // 2)


def _unsplit(o):
    return o[:, :R].transpose(1, 0, 2).reshape(R, -1)


def kernel(x, edge_index, marks, edge_x, edge_marks,
           Wn0, bn0, We0, be0, Wn1, bn1, We1, be1,
           Wn2, bn2, We2, be2, Wn3, bn3, We3, be3,
           W1, b1, W2, b2):
    ni = edge_index[0]
    ei = edge_index[1]

    npad = EP - E
    pad_src = (jnp.arange(npad, dtype=jnp.int32) % R)
    pad_dst = R + (jnp.arange(npad, dtype=jnp.int32) % (RR - R))

    src_n = jnp.concatenate([ni, pad_src])
    src_e = jnp.concatenate([ei, pad_src])
    srcn2 = jnp.stack([src_n, src_n + RR]).reshape(NC, NROWS, 128)
    srce2 = jnp.stack([src_e, src_e + RR]).reshape(NC, NROWS, 128)
    dst_n = jnp.concatenate([ni, pad_dst]).reshape(NROWS, 128)
    dst_e = jnp.concatenate([ei, pad_dst]).reshape(NROWS, 128)
    dst_both = jnp.stack([dst_n, dst_e])

    ones_h = jnp.ones((128, 16), jnp.float32)
    z32 = jnp.zeros((RPT, 32), jnp.float32)
    z16 = jnp.zeros((RPT, 16), jnp.float32)

    degs = _deg_kernel(dst_both, ones_h, z16)
    dn = degs[0, :R, 0]
    db = degs[1, :R, 0]
    dinv = jnp.where(dn > 0, 1.0 / dn, 0.0)
    binv = jnp.where(db > 0, 1.0 / db, 0.0)

    def spmm(table_split, srcs, dsts, fh):
        f = _spmm32 if fh == 32 else _spmm16
        return f(table_split, srcs, dsts, z32 if fh == 32 else z16)

    Wns = [(Wn0, bn0), (Wn1, bn1), (Wn2, bn2), (Wn3, bn3)]
    Wes = [(We0, be0), (We1, be1), (We2, be2), (We3, be3)]
    all_x, all_e = [], []
    cur, cure = x, edge_x
    for (Wn, bn), (We, be) in zip(Wns, Wes):
        xw = cur @ Wn + bn
        s1 = _unsplit(spmm(_split(xw), srcn2, dst_e, 32))
        emsg = s1 * binv[:, None]
        s2 = _unsplit(spmm(_split(emsg), srce2, dst_n, 32))
        cur = jax.nn.relu(s2 * dinv[:, None])
        all_x.append(cur)
        ew = cure @ We + be
        t1 = _unsplit(spmm(_split(ew), srce2, dst_n, 16))
        nmsg = t1 * dinv[:, None]
        t2 = _unsplit(spmm(_split(nmsg), srcn2, dst_e, 16))
        cure = jax.nn.relu(t2 * binv[:, None])
        all_e.append(cure)

    xc = jnp.concatenate(all_x, axis=1)[marks]
    ec = jnp.concatenate(all_e, axis=1)
    e1 = ec[edge_marks]
    e2 = ec[edge_marks + 1]
    ex = jnp.concatenate([jnp.minimum(e1, e2), jnp.maximum(e1, e2)], axis=1)
    h = jnp.concatenate([ex, xc], axis=1)
    return _head(h, W1, b1, W2, b2)


# speedup vs baseline: 7.5764x; 1.4741x over previous
import functools

import jax
import jax.numpy as jnp
from jax import lax
from jax.experimental import pallas as pl
from jax.experimental.pallas import tpu as pltpu
from jax.experimental.pallas import tpu_sc as plsc

NC = 2
NS = 16
R = 50000
RR = 50048
E = 800000
EP = 819200
NROWS = EP // 128
TROWS = NROWS // NS
CHROWS = 4
NCHUNK = TROWS // CHROWS
RPT = RR // NS

_mesh = plsc.VectorSubcoreMesh(
    core_axis_name="c", subcore_axis_name="s", num_cores=NC, num_subcores=NS)


NBUF = 4
GR = 8
NGRP = TROWS // GR


def _spmm_body(table, srcs, dsts, zrows, out,
               gsrc, gdst, sbuf, dbuf, rows_v, acc,
               g0, g1, g2, g3, s0, s1, s2, s3):
    c = lax.axis_index("c")
    s = lax.axis_index("s")
    gsem = [g0, g1, g2, g3]
    ssem = [s0, s1, s2, s3]
    fh = rows_v.shape[1]
    pltpu.sync_copy(zrows, acc.at[pl.ds(s * RPT, RPT)])
    plsc.subcore_barrier()

    t0 = s * TROWS

    def gat(b):
        return pltpu.make_async_copy(
            table.at[sbuf.at[b]], rows_v.at[pl.ds(b * 128, 128)], gsem[b])

    def sca(b):
        return pltpu.make_async_copy(
            rows_v.at[pl.ds(b * 128, 128)], acc.at[dbuf.at[b]], ssem[b])

    def stage_and_fire(n, b):
        for i in range(8):
            sbuf[b, pl.ds(i * 16, 16)] = gsrc[n, pl.ds(i * 16, 16)]
            dbuf[b, pl.ds(i * 16, 16)] = gdst[n, pl.ds(i * 16, 16)]
        gat(b).start()

    def consume(b):
        gat(b).wait()
        sca(b).start(add=True)

    def group(g, _):
        r0 = t0 + g * GR
        pltpu.sync_copy(srcs.at[c, pl.ds(r0, GR)], gsrc)
        pltpu.sync_copy(dsts.at[pl.ds(r0, GR)], gdst)
        for n in range(GR):
            b = n % NBUF
            if n >= NBUF:
                sca(b).wait()
            else:
                @pl.when(g > 0)
                def _():
                    sca(b).wait()
            stage_and_fire(n, b)
            m = n - (NBUF - 1)
            bm = m % NBUF
            if m >= 0:
                consume(bm)
            else:
                @pl.when(g > 0)
                def _():
                    consume(bm)
        return 0

    lax.fori_loop(0, NGRP, group, 0)
    for n in range(GR - (NBUF - 1), GR):
        consume(n % NBUF)
    for b in range(NBUF):
        sca(b).wait()
    plsc.subcore_barrier()
    pltpu.sync_copy(acc.at[pl.ds(s * RPT, RPT)],
                    out.at[c, pl.ds(s * RPT, RPT)])


def _make_spmm(fh):
    return pl.kernel(
        _spmm_body,
        out_type=jax.ShapeDtypeStruct((NC, RR, fh), jnp.float32),
        mesh=_mesh,
        compiler_params=pltpu.CompilerParams(use_tc_tiling_on_sc=False),
        scratch_types=[
            pltpu.VMEM((GR, 128), jnp.int32),
            pltpu.VMEM((GR, 128), jnp.int32),
            pltpu.VMEM((NBUF, 128), jnp.int32),
            pltpu.VMEM((NBUF, 128), jnp.int32),
            pltpu.VMEM((NBUF * 128, fh), jnp.float32),
            pltpu.VMEM_SHARED((RR, fh), jnp.float32),
        ] + [pltpu.SemaphoreType.DMA] * 8,
    )


_spmm32 = _make_spmm(32)
_spmm16 = _make_spmm(16)


def _deg_body(dsts, ones_h, zrows, out, dst_v, ones_v, acc, sem):
    c = lax.axis_index("c")
    s = lax.axis_index("s")
    pltpu.sync_copy(zrows, acc.at[pl.ds(s * RPT, RPT)])
    pltpu.sync_copy(ones_h, ones_v)
    plsc.subcore_barrier()

    t0 = s * TROWS

    def chunk(k, _):
        r0 = t0 + k * CHROWS
        pltpu.sync_copy(dsts.at[c, pl.ds(r0, CHROWS)], dst_v)
        for j in range(CHROWS):
            pltpu.sync_copy(ones_v, acc.at[dst_v.at[j]], add=True)
        return 0

    lax.fori_loop(0, NCHUNK, chunk, 0)
    plsc.subcore_barrier()
    pltpu.sync_copy(acc.at[pl.ds(s * RPT, RPT)],
                    out.at[c, pl.ds(s * RPT, RPT)])


_deg_kernel = pl.kernel(
    _deg_body,
    out_type=jax.ShapeDtypeStruct((NC, RR, 16), jnp.float32),
    mesh=_mesh,
    compiler_params=pltpu.CompilerParams(use_tc_tiling_on_sc=False),
    scratch_types=[
        pltpu.VMEM((CHROWS, 128), jnp.int32),
        pltpu.VMEM((128, 16), jnp.float32),
        pltpu.VMEM_SHARED((RR, 16), jnp.float32),
        pltpu.SemaphoreType.DMA,
    ],
)


def _head_kernel_body(h_ref, w1_ref, b1_ref, w2_ref, b2_ref, out_ref):
    h = h_ref[...]
    z = jnp.maximum(h @ w1_ref[...] + b1_ref[...][None, :], 0.0)
    logits = z @ w2_ref[...] + b2_ref[...][None, :]
    m = jnp.max(logits, axis=1, keepdims=True)
    sm = jnp.log(jnp.sum(jnp.exp(logits - m), axis=1, keepdims=True)) + m
    out_ref[...] = logits - sm


def _head(h, W1, b1, W2, b2):
    return pl.pallas_call(
        _head_kernel_body,
        out_shape=jax.ShapeDtypeStruct((h.shape[0], 2), jnp.float32),
    )(h, W1, b1, W2, b2)


def _split(t):
    n, f = t.shape
    th = t.reshape(n, 2, f // 2).transpose(1, 0, 2)
    th = jnp.pad(th, ((0, 0), (0, RR - n), (0, 0)))
    return th.reshape(2 * RR, f // 2)


def _unsplit(o):
    return o[:, :R].transpose(1, 0, 2).reshape(R, -1)


def kernel(x, edge_index, marks, edge_x, edge_marks,
           Wn0, bn0, We0, be0, Wn1, bn1, We1, be1,
           Wn2, bn2, We2, be2, Wn3, bn3, We3, be3,
           W1, b1, W2, b2):
    ni = edge_index[0]
    ei = edge_index[1]

    npad = EP - E
    pad_src = (jnp.arange(npad, dtype=jnp.int32) % R)
    pad_dst = R + (jnp.arange(npad, dtype=jnp.int32) % (RR - R))

    src_n = jnp.concatenate([ni, pad_src])
    src_e = jnp.concatenate([ei, pad_src])
    srcn2 = jnp.stack([src_n, src_n + RR]).reshape(NC, NROWS, 128)
    srce2 = jnp.stack([src_e, src_e + RR]).reshape(NC, NROWS, 128)
    dst_n = jnp.concatenate([ni, pad_dst]).reshape(NROWS, 128)
    dst_e = jnp.concatenate([ei, pad_dst]).reshape(NROWS, 128)
    dst_both = jnp.stack([dst_n, dst_e])

    ones_h = jnp.ones((128, 16), jnp.float32)
    z32 = jnp.zeros((RPT, 32), jnp.float32)
    z16 = jnp.zeros((RPT, 16), jnp.float32)

    degs = _deg_kernel(dst_both, ones_h, z16)
    dn = degs[0, :R, 0]
    db = degs[1, :R, 0]
    dinv = jnp.where(dn > 0, 1.0 / dn, 0.0)
    binv = jnp.where(db > 0, 1.0 / db, 0.0)

    def spmm(table_split, srcs, dsts, fh):
        f = _spmm32 if fh == 32 else _spmm16
        return f(table_split, srcs, dsts, z32 if fh == 32 else z16)

    Wns = [(Wn0, bn0), (Wn1, bn1), (Wn2, bn2), (Wn3, bn3)]
    Wes = [(We0, be0), (We1, be1), (We2, be2), (We3, be3)]
    all_x, all_e = [], []
    cur, cure = x, edge_x
    for (Wn, bn), (We, be) in zip(Wns, Wes):
        xw = cur @ Wn + bn
        s1 = _unsplit(spmm(_split(xw), srcn2, dst_e, 32))
        emsg = s1 * binv[:, None]
        s2 = _unsplit(spmm(_split(emsg), srce2, dst_n, 32))
        cur = jax.nn.relu(s2 * dinv[:, None])
        all_x.append(cur)
        ew = cure @ We + be
        t1 = _unsplit(spmm(_split(ew), srce2, dst_n, 16))
        nmsg = t1 * dinv[:, None]
        t2 = _unsplit(spmm(_split(nmsg), srcn2, dst_e, 16))
        cure = jax.nn.relu(t2 * binv[:, None])
        all_e.append(cure)

    xc = jnp.concatenate(all_x, axis=1)[marks]
    ec = jnp.concatenate(all_e, axis=1)
    e1 = ec[edge_marks]
    e2 = ec[edge_marks + 1]
    ex = jnp.concatenate([jnp.minimum(e1, e2), jnp.maximum(e1, e2)], axis=1)
    h = jnp.concatenate([ex, xc], axis=1)
    return _head(h, W1, b1, W2, b2)


# speedup vs baseline: 7.9987x; 1.0557x over previous
import jax
import jax.numpy as jnp
from jax import lax
from jax.experimental import pallas as pl
from jax.experimental.pallas import tpu as pltpu
from jax.experimental.pallas import tpu_sc as plsc

NC = 2
NS = 16
R = 50000
RR = 50048
E = 800000
EP = 819200
NROWS = EP // 128
TROWS = NROWS // NS
RPT = RR // NS
NBUF = 4
GR = 8
NGRP = TROWS // GR
BR = RR // 16

_mesh = plsc.VectorSubcoreMesh(
    core_axis_name="c", subcore_axis_name="s", num_cores=NC, num_subcores=NS)
_sc_params = pltpu.CompilerParams(use_tc_tiling_on_sc=False)


def _spmm_body(table, srcs, dsts, zrows, out,
               gsrc, gdst, sbuf, dbuf, rows_v, acc,
               g0, g1, g2, g3, s0, s1, s2, s3):
    c = lax.axis_index("c")
    s = lax.axis_index("s")
    gsem = [g0, g1, g2, g3]
    ssem = [s0, s1, s2, s3]
    pltpu.sync_copy(zrows, acc.at[pl.ds(s * RPT, RPT)])
    plsc.subcore_barrier()

    t0 = s * TROWS

    def gat(b):
        return pltpu.make_async_copy(
            table.at[sbuf.at[b]], rows_v.at[pl.ds(b * 128, 128)], gsem[b])

    def sca(b):
        return pltpu.make_async_copy(
            rows_v.at[pl.ds(b * 128, 128)], acc.at[dbuf.at[b]], ssem[b])

    def stage_and_fire(n, b):
        for i in range(8):
            sbuf[b, pl.ds(i * 16, 16)] = gsrc[n, pl.ds(i * 16, 16)]
            dbuf[b, pl.ds(i * 16, 16)] = gdst[n, pl.ds(i * 16, 16)]
        gat(b).start()

    def consume(b):
        gat(b).wait()
        sca(b).start(add=True)

    def group(g, _):
        r0 = t0 + g * GR
        pltpu.sync_copy(srcs.at[c, pl.ds(r0, GR)], gsrc)
        pltpu.sync_copy(dsts.at[pl.ds(r0, GR)], gdst)
        for n in range(GR):
            b = n % NBUF
            if n >= NBUF:
                sca(b).wait()
            else:
                @pl.when(g > 0)
                def _():
                    sca(b).wait()
            stage_and_fire(n, b)
            m = n - (NBUF - 1)
            bm = m % NBUF
            if m >= 0:
                consume(bm)
            else:
                @pl.when(g > 0)
                def _():
                    consume(bm)
        return 0

    lax.fori_loop(0, NGRP, group, 0)
    for n in range(GR - (NBUF - 1), GR):
        consume(n % NBUF)
    for b in range(NBUF):
        sca(b).wait()
    plsc.subcore_barrier()
    pltpu.sync_copy(acc.at[pl.ds(s * RPT, RPT)],
                    out.at[c, pl.ds(s * RPT, RPT)])


def _make_spmm(fh):
    return pl.kernel(
        _spmm_body,
        out_type=jax.ShapeDtypeStruct((NC, RR, fh), jnp.float32),
        mesh=_mesh,
        compiler_params=_sc_params,
        scratch_types=[
            pltpu.VMEM((GR, 128), jnp.int32),
            pltpu.VMEM((GR, 128), jnp.int32),
            pltpu.VMEM((NBUF, 128), jnp.int32),
            pltpu.VMEM((NBUF, 128), jnp.int32),
            pltpu.VMEM((NBUF * 128, fh), jnp.float32),
            pltpu.VMEM_SHARED((RR, fh), jnp.float32),
        ] + [pltpu.SemaphoreType.DMA] * 8,
    )


_spmm32 = _make_spmm(32)
_spmm16 = _make_spmm(16)


def _deg_body(dsts, ones_h, zrows, out, dst_v, ones_v, acc, sem):
    c = lax.axis_index("c")
    s = lax.axis_index("s")
    pltpu.sync_copy(zrows, acc.at[pl.ds(s * RPT, RPT)])
    pltpu.sync_copy(ones_h, ones_v)
    plsc.subcore_barrier()

    t0 = s * TROWS

    def chunk(k, _):
        r0 = t0 + k * 4
        pltpu.sync_copy(dsts.at[c, pl.ds(r0, 4)], dst_v)
        for j in range(4):
            pltpu.sync_copy(ones_v, acc.at[dst_v.at[j]], add=True)
        return 0

    lax.fori_loop(0, TROWS // 4, chunk, 0)
    plsc.subcore_barrier()
    pltpu.sync_copy(acc.at[pl.ds(s * RPT, RPT)],
                    out.at[c, pl.ds(s * RPT, RPT)])


_deg_kernel = pl.kernel(
    _deg_body,
    out_type=jax.ShapeDtypeStruct((NC, RR, 16), jnp.float32),
    mesh=_mesh,
    compiler_params=_sc_params,
    scratch_types=[
        pltpu.VMEM((4, 128), jnp.int32),
        pltpu.VMEM((128, 16), jnp.float32),
        pltpu.VMEM_SHARED((RR, 16), jnp.float32),
        pltpu.SemaphoreType.DMA,
    ],
)


def _headgather_body(n0, n1, n2, n3, e0, e1, e2, e3, mflat, emflat, emflat1,
                     xc, e1c, e2c, mk_v, nbuf, ebuf, sem):
    c = lax.axis_index("c")
    s = lax.axis_index("s")
    w = s * NC + c
    ntabs = [n0, n1, n2, n3]
    etabs = [e0, e1, e2, e3]
    r0 = w * 128
    for i in range(4):
        for hh in range(2):
            pltpu.sync_copy(mflat.at[2 * w + hh], mk_v)
            pltpu.async_copy(ntabs[i].at[mk_v], nbuf, sem).wait()
            pltpu.sync_copy(
                nbuf, xc.at[pl.ds(r0, 128), pl.ds(i * 64 + hh * 32, 32)])
            pltpu.sync_copy(emflat.at[2 * w + hh], mk_v)
            pltpu.async_copy(etabs[i].at[mk_v], ebuf, sem).wait()
            pltpu.sync_copy(
                ebuf, e1c.at[pl.ds(r0, 128), pl.ds(i * 32 + hh * 16, 16)])
            pltpu.sync_copy(emflat1.at[2 * w + hh], mk_v)
            pltpu.async_copy(etabs[i].at[mk_v], ebuf, sem).wait()
            pltpu.sync_copy(
                ebuf, e2c.at[pl.ds(r0, 128), pl.ds(i * 32 + hh * 16, 16)])


_headgather = pl.kernel(
    _headgather_body,
    out_type=(jax.ShapeDtypeStruct((4096, 256), jnp.float32),
              jax.ShapeDtypeStruct((4096, 128), jnp.float32),
              jax.ShapeDtypeStruct((4096, 128), jnp.float32)),
    mesh=_mesh,
    compiler_params=_sc_params,
    scratch_types=[
        pltpu.VMEM((128,), jnp.int32),
        pltpu.VMEM((128, 32), jnp.float32),
        pltpu.VMEM((128, 16), jnp.float32),
        pltpu.SemaphoreType.DMA,
    ],
)


def _mm0_body(x_ref, ex_ref, wn_ref, bn_ref, we_ref, be_ref,
              xwn_ref, xwe_ref):
    a = x_ref[...] @ wn_ref[...] + bn_ref[...]
    xwn_ref[0] = a[:, :32]
    xwn_ref[1] = a[:, 32:]
    ae = ex_ref[...] @ we_ref[...] + be_ref[...]
    xwe_ref[0] = ae[:, :16]
    xwe_ref[1] = ae[:, 16:]


def _mm0(x, ex, Wn, bn, We, be):
    return pl.pallas_call(
        _mm0_body,
        grid=(16,),
        in_specs=[
            pl.BlockSpec((BR, 128), lambda i: (i, 0)),
            pl.BlockSpec((BR, 64), lambda i: (i, 0)),
            pl.BlockSpec((128, 64), lambda i: (0, 0)),
            pl.BlockSpec((1, 64), lambda i: (0, 0)),
            pl.BlockSpec((64, 32), lambda i: (0, 0)),
            pl.BlockSpec((1, 32), lambda i: (0, 0)),
        ],
        out_specs=[
            pl.BlockSpec((2, BR, 32), lambda i: (0, i, 0)),
            pl.BlockSpec((2, BR, 16), lambda i: (0, i, 0)),
        ],
        out_shape=[
            jax.ShapeDtypeStruct((2, RR, 32), jnp.float32),
            jax.ShapeDtypeStruct((2, RR, 16), jnp.float32),
        ],
    )(x, ex, Wn, bn[None, :], We, be[None, :])


def _invcols(degs_blk):
    dn = degs_blk[0, :, 0:1]
    db = degs_blk[1, :, 0:1]
    dinv = jnp.where(dn > 0, 1.0 / dn, 0.0)
    binv = jnp.where(db > 0, 1.0 / db, 0.0)
    return dinv, binv


def _midscale_body(s1n_ref, t1e_ref, degs_ref, emsg_ref, nmsg_ref):
    dinv, binv = _invcols(degs_ref[...])
    emsg_ref[0] = s1n_ref[0] * binv
    emsg_ref[1] = s1n_ref[1] * binv
    nmsg_ref[0] = t1e_ref[0] * dinv
    nmsg_ref[1] = t1e_ref[1] * dinv


def _midscale(s1n, t1e, degs):
    return pl.pallas_call(
        _midscale_body,
        grid=(16,),
        in_specs=[
            pl.BlockSpec((2, BR, 32), lambda i: (0, i, 0)),
            pl.BlockSpec((2, BR, 16), lambda i: (0, i, 0)),
            pl.BlockSpec((2, BR, 16), lambda i: (0, i, 0)),
        ],
        out_specs=[
            pl.BlockSpec((2, BR, 32), lambda i: (0, i, 0)),
            pl.BlockSpec((2, BR, 16), lambda i: (0, i, 0)),
        ],
        out_shape=[
            jax.ShapeDtypeStruct((2, RR, 32), jnp.float32),
            jax.ShapeDtypeStruct((2, RR, 16), jnp.float32),
        ],
    )(s1n, t1e, degs)


def _layer_body(s2n_ref, t2e_ref, degs_ref, wn_ref, bn_ref, we_ref, be_ref,
                actn_ref, acte_ref, xwn_ref, xwe_ref):
    dinv, binv = _invcols(degs_ref[...])
    a0 = jnp.maximum(s2n_ref[0] * dinv, 0.0)
    a1 = jnp.maximum(s2n_ref[1] * dinv, 0.0)
    actn_ref[0] = a0
    actn_ref[1] = a1
    xa = jnp.concatenate([a0, a1], axis=1) @ wn_ref[...] + bn_ref[...]
    xwn_ref[0] = xa[:, :32]
    xwn_ref[1] = xa[:, 32:]
    b0 = jnp.maximum(t2e_ref[0] * binv, 0.0)
    b1 = jnp.maximum(t2e_ref[1] * binv, 0.0)
    acte_ref[0] = b0
    acte_ref[1] = b1
    xe = jnp.concatenate([b0, b1], axis=1) @ we_ref[...] + be_ref[...]
    xwe_ref[0] = xe[:, :16]
    xwe_ref[1] = xe[:, 16:]


def _layer_mm(s2n, t2e, degs, Wn, bn, We, be):
    return pl.pallas_call(
        _layer_body,
        grid=(16,),
        in_specs=[
            pl.BlockSpec((2, BR, 32), lambda i: (0, i, 0)),
            pl.BlockSpec((2, BR, 16), lambda i: (0, i, 0)),
            pl.BlockSpec((2, BR, 16), lambda i: (0, i, 0)),
            pl.BlockSpec((64, 64), lambda i: (0, 0)),
            pl.BlockSpec((1, 64), lambda i: (0, 0)),
            pl.BlockSpec((32, 32), lambda i: (0, 0)),
            pl.BlockSpec((1, 32), lambda i: (0, 0)),
        ],
        out_specs=[
            pl.BlockSpec((2, BR, 32), lambda i: (0, i, 0)),
            pl.BlockSpec((2, BR, 16), lambda i: (0, i, 0)),
            pl.BlockSpec((2, BR, 32), lambda i: (0, i, 0)),
            pl.BlockSpec((2, BR, 16), lambda i: (0, i, 0)),
        ],
        out_shape=[
            jax.ShapeDtypeStruct((2, RR, 32), jnp.float32),
            jax.ShapeDtypeStruct((2, RR, 16), jnp.float32),
            jax.ShapeDtypeStruct((2, RR, 32), jnp.float32),
            jax.ShapeDtypeStruct((2, RR, 16), jnp.float32),
        ],
    )(s2n, t2e, degs, Wn, bn[None, :], We, be[None, :])


def _final_body(s2n_ref, t2e_ref, degs_ref, actn_ref, acte_ref):
    dinv, binv = _invcols(degs_ref[...])
    actn_ref[0] = jnp.maximum(s2n_ref[0] * dinv, 0.0)
    actn_ref[1] = jnp.maximum(s2n_ref[1] * dinv, 0.0)
    acte_ref[0] = jnp.maximum(t2e_ref[0] * binv, 0.0)
    acte_ref[1] = jnp.maximum(t2e_ref[1] * binv, 0.0)


def _final_scale(s2n, t2e, degs):
    return pl.pallas_call(
        _final_body,
        grid=(16,),
        in_specs=[
            pl.BlockSpec((2, BR, 32), lambda i: (0, i, 0)),
            pl.BlockSpec((2, BR, 16), lambda i: (0, i, 0)),
            pl.BlockSpec((2, BR, 16), lambda i: (0, i, 0)),
        ],
        out_specs=[
            pl.BlockSpec((2, BR, 32), lambda i: (0, i, 0)),
            pl.BlockSpec((2, BR, 16), lambda i: (0, i, 0)),
        ],
        out_shape=[
            jax.ShapeDtypeStruct((2, RR, 32), jnp.float32),
            jax.ShapeDtypeStruct((2, RR, 16), jnp.float32),
        ],
    )(s2n, t2e, degs)


def _head_body(e1_ref, e2_ref, xc_ref, w1_ref, b1_ref, w2_ref, b2_ref,
               out_ref):
    e1 = e1_ref[...]
    e2 = e2_ref[...]
    h = jnp.concatenate(
        [jnp.minimum(e1, e2), jnp.maximum(e1, e2), xc_ref[...]], axis=1)
    z = jnp.maximum(h @ w1_ref[...] + b1_ref[...], 0.0)
    logits = z @ w2_ref[...] + b2_ref[...]
    m = jnp.max(logits, axis=1, keepdims=True)
    sm = jnp.log(jnp.sum(jnp.exp(logits - m), axis=1, keepdims=True)) + m
    out_ref[...] = logits - sm


def _head(e1c, e2c, xc, W1, b1, W2, b2):
    return pl.pallas_call(
        _head_body,
        out_shape=jax.ShapeDtypeStruct((4096, 2), jnp.float32),
    )(e1c, e2c, xc, W1, b1[None, :], W2, b2[None, :])


def kernel(x, edge_index, marks, edge_x, edge_marks,
           Wn0, bn0, We0, be0, Wn1, bn1, We1, be1,
           Wn2, bn2, We2, be2, Wn3, bn3, We3, be3,
           W1, b1, W2, b2):
    ni = edge_index[0]
    ei = edge_index[1]

    npad = EP - E
    pad_src = jnp.arange(npad, dtype=jnp.int32) % R
    pad_dst = R + (jnp.arange(npad, dtype=jnp.int32) % (RR - R))

    src_n = jnp.concatenate([ni, pad_src])
    src_e = jnp.concatenate([ei, pad_src])
    srcn2 = jnp.stack([src_n, src_n + RR]).reshape(NC, NROWS, 128)
    srce2 = jnp.stack([src_e, src_e + RR]).reshape(NC, NROWS, 128)
    dst_n = jnp.concatenate([ni, pad_dst]).reshape(NROWS, 128)
    dst_e = jnp.concatenate([ei, pad_dst]).reshape(NROWS, 128)
    dst_both = jnp.stack([dst_n, dst_e])

    m2 = marks.reshape(32, 128)
    mflat = jnp.stack([m2, m2 + RR], axis=1).reshape(64, 128)
    em2 = edge_marks.reshape(32, 128)
    emflat = jnp.stack([em2, em2 + RR], axis=1).reshape(64, 128)
    em2p = em2 + 1
    emflat1 = jnp.stack([em2p, em2p + RR], axis=1).reshape(64, 128)

    ones_h = jnp.ones((128, 16), jnp.float32)
    z32 = jnp.zeros((RPT, 32), jnp.float32)
    z16 = jnp.zeros((RPT, 16), jnp.float32)

    degs = _deg_kernel(dst_both, ones_h, z16)
    xwn, xwe = _mm0(x, edge_x, Wn0, bn0, We0, be0)

    Ws = [(Wn1, bn1, We1, be1), (Wn2, bn2, We2, be2), (Wn3, bn3, We3, be3)]
    all_n, all_e = [], []
    for i in range(4):
        s1n = _spmm32(xwn.reshape(2 * RR, 32), srcn2, dst_e, z32)
        t1e = _spmm16(xwe.reshape(2 * RR, 16), srce2, dst_n, z16)
        emsg, nmsg = _midscale(s1n, t1e, degs)
        s2n = _spmm32(emsg.reshape(2 * RR, 32), srce2, dst_n, z32)
        t2e = _spmm16(nmsg.reshape(2 * RR, 16), srcn2, dst_e, z16)
        if i < 3:
            Wn, bn, We, be = Ws[i]
            actn, acte, xwn, xwe = _layer_mm(s2n, t2e, degs, Wn, bn, We, be)
        else:
            actn, acte = _final_scale(s2n, t2e, degs)
        all_n.append(actn)
        all_e.append(acte)

    xc, e1c, e2c = _headgather(
        all_n[0].reshape(2 * RR, 32), all_n[1].reshape(2 * RR, 32),
        all_n[2].reshape(2 * RR, 32), all_n[3].reshape(2 * RR, 32),
        all_e[0].reshape(2 * RR, 16), all_e[1].reshape(2 * RR, 16),
        all_e[2].reshape(2 * RR, 16), all_e[3].reshape(2 * RR, 16),
        mflat, emflat, emflat1)
    return _head(e1c, e2c, xc, W1, b1, W2, b2)
